# Initial kernel scaffold; baseline (speedup 1.0000x reference)
import jax
import jax.numpy as jnp
from jax.experimental import pallas as pl


def kernel(features, preference, W1, b1, W2, b2, edge_index):
    raise NotImplementedError("write your pallas kernel here")


# speedup vs baseline: 9.8007x; 9.8007x over previous
import functools

import jax
import jax.numpy as jnp
from jax import lax
from jax.experimental import pallas as pl
from jax.experimental.pallas import tpu as pltpu
from jax.experimental.pallas import tpu_sc as plsc

N_USER = 25000
N_ITEM = 25000
N = N_USER + N_ITEM
D = 64
H = 32
NPAD = 51200
E = 800000
EP = 802816
RPT = NPAD // 16
CW = 128


SEG = NPAD // 16


def _sca_body(rowb, zerosf, hists, degp, hist, rowbuf, sbuf):
    c = lax.axis_index("c")
    s = lax.axis_index("s")
    pltpu.sync_copy(zerosf, hist)
    ones = jnp.ones((16,), jnp.float32)

    def hgroup(gi, carry):
        pltpu.sync_copy(rowb.at[s, gi], rowbuf)
        for j in range(8):
            for g in range(8):
                idx = rowbuf[j, pl.ds(g * 16, 16)]
                plsc.addupdate_scatter(hist, [idx], ones)
        return carry

    lax.fori_loop(c * 25, 25 + c * 24, hgroup, 0)

    pltpu.sync_copy(hist, hists.at[c, s])
    plsc.subcore_barrier()
    for t in range(16):
        pltpu.sync_copy(hists.at[c, t, pl.ds(s * SEG, SEG)], sbuf.at[t])

    def addg(g, carry):
        tot = sbuf[0, pl.ds(g * 16, 16)]
        for t in range(1, 16):
            tot = tot + sbuf[t, pl.ds(g * 16, 16)]
        hist[pl.ds(g * 16, 16)] = tot
        return carry

    lax.fori_loop(0, SEG // 16, addg, 0)
    pltpu.sync_copy(hist.at[pl.ds(0, SEG)], degp.at[c, pl.ds(s * SEG, SEG)])


def _degree(rowb, zerosf):
    mesh = plsc.VectorSubcoreMesh(core_axis_name="c", subcore_axis_name="s")
    return pl.kernel(
        _sca_body,
        out_type=(
            jax.ShapeDtypeStruct((2, 16, NPAD), jnp.float32),
            jax.ShapeDtypeStruct((2, NPAD), jnp.float32),
        ),
        mesh=mesh,
        scratch_types=[
            pltpu.VMEM((NPAD,), jnp.float32),
            pltpu.VMEM((8, 128), jnp.int32),
            pltpu.VMEM((16, SEG), jnp.float32),
        ],
        compiler_params=pltpu.CompilerParams(needs_layout_passes=False, use_tc_tiling_on_sc=False),
    )(rowb, zerosf)


def _mlp_body(x_ref, w1_ref, b1_ref, w2_ref, b2_ref, o_ref):
    h = lax.dot_general(x_ref[...], w1_ref[...],
                        (((1,), (1,)), ((), ())),
                        preferred_element_type=jnp.float32)
    h = h + b1_ref[...]
    h = jnp.where(h >= 0, h, 0.01 * h)
    o = lax.dot_general(h, w2_ref[...],
                        (((1,), (1,)), ((), ())),
                        preferred_element_type=jnp.float32)
    o_ref[...] = o + b2_ref[...]


def _mlp(features, W1, b1, W2, b2):
    bm = 1000
    return pl.pallas_call(
        _mlp_body,
        grid=(N_ITEM // bm,),
        in_specs=[
            pl.BlockSpec((bm, 128), lambda i: (i, 0)),
            pl.BlockSpec((256, 128), lambda i: (0, 0)),
            pl.BlockSpec((1, 256), lambda i: (0, 0)),
            pl.BlockSpec((64, 256), lambda i: (0, 0)),
            pl.BlockSpec((1, 64), lambda i: (0, 0)),
        ],
        out_specs=pl.BlockSpec((bm, 64), lambda i: (i, 0)),
        out_shape=jax.ShapeDtypeStruct((N_ITEM, 64), jnp.float32),
    )(features, W1, b1.reshape(1, 256), W2, b2.reshape(1, 64))


def _table_body(ego_ref, degc_ref, table_ref, disx_ref):
    deg = degc_ref[0] + degc_ref[1]
    dis = jnp.where(deg > 0, lax.rsqrt(jnp.maximum(deg, 1e-12)), 0.0)
    disx = jnp.broadcast_to(dis, (CW, H))
    disx_ref[...] = disx
    table_ref[...] = ego_ref[0] * disx


def _table0(egoT, degc):
    nb = NPAD // CW
    return pl.pallas_call(
        _table_body,
        grid=(2, nb),
        in_specs=[
            pl.BlockSpec((1, CW, H), lambda c, i: (c, i, 0)),
            pl.BlockSpec((2, CW, 1), lambda c, i: (0, i, 0)),
        ],
        out_specs=[
            pl.BlockSpec((CW, H), lambda c, i: (c * nb + i, 0)),
            pl.BlockSpec((CW, H), lambda c, i: (i, 0)),
        ],
        out_shape=[
            jax.ShapeDtypeStruct((2 * NPAD, H), jnp.float32),
            jax.ShapeDtypeStruct((NPAD, H), jnp.float32),
        ],
    )(egoT, degc)


def _scb_body(table0, colb, rowb, disx, egoT, zerosa,
              out, table1, fsum,
              acc, colbuf, rowbuf, gbuf, accbuf, e0buf, disbuf, sem):
    c = lax.axis_index("c")
    s = lax.axis_index("s")

    def edge_pass(table_ref):
        def group(gi, carry):
            pltpu.sync_copy(colb.at[c, s, gi], colbuf)
            pltpu.sync_copy(rowb.at[s, gi], rowbuf)
            for j in range(8):
                pltpu.async_copy(table_ref.at[colbuf.at[j]], gbuf, sem).wait()
                pltpu.sync_copy(gbuf, acc.at[rowbuf.at[j]], add=True)
            return carry

        lax.fori_loop(0, 49, group, 0)

    def writeback(layer):
        for ch in range(RPT // CW):
            b = s * RPT + ch * CW
            pltpu.sync_copy(acc.at[pl.ds(b, CW), :], accbuf)
            if layer == 1:
                pltpu.sync_copy(egoT.at[c, pl.ds(b, CW), :], e0buf)
            else:
                pltpu.sync_copy(fsum.at[c, pl.ds(b, CW), :], e0buf)
            pltpu.sync_copy(disx.at[pl.ds(b, CW), :], disbuf)

            def rowfn(r, carry):
                for g in range(2):
                    dv = disbuf[r, pl.ds(g * 16, 16)]
                    a = accbuf[r, pl.ds(g * 16, 16)] * dv
                    f = e0buf[r, pl.ds(g * 16, 16)] + a
                    if layer == 1:
                        e0buf[r, pl.ds(g * 16, 16)] = f
                        accbuf[r, pl.ds(g * 16, 16)] = a * dv
                    else:
                        e0buf[r, pl.ds(g * 16, 16)] = f * (1.0 / 3.0)
                return carry

            lax.fori_loop(0, CW, rowfn, 0)
            if layer == 1:
                pltpu.sync_copy(e0buf, fsum.at[c, pl.ds(b, CW), :])
                pltpu.sync_copy(accbuf, table1.at[pl.ds(c * NPAD + b, CW), :])
                pltpu.sync_copy(zerosa.at[pl.ds(0, CW), :],
                                acc.at[pl.ds(b, CW), :])
            else:
                pltpu.sync_copy(e0buf, out.at[c, pl.ds(b, CW), :])

    pltpu.sync_copy(zerosa, acc.at[pl.ds(s * RPT, RPT), :])
    plsc.subcore_barrier()
    edge_pass(table0)
    plsc.subcore_barrier()
    writeback(1)
    plsc.subcore_barrier()
    edge_pass(table1)
    plsc.subcore_barrier()
    writeback(2)


def _propagate(table0, colb, rowb, disx, egoT, zerosa):
    mesh = plsc.VectorSubcoreMesh(core_axis_name="c", subcore_axis_name="s")
    return pl.kernel(
        _scb_body,
        out_type=(
            jax.ShapeDtypeStruct((2, NPAD, H), jnp.float32),
            jax.ShapeDtypeStruct((2 * NPAD, H), jnp.float32),
            jax.ShapeDtypeStruct((2, NPAD, H), jnp.float32),
        ),
        mesh=mesh,
        scratch_types=[
            pltpu.VMEM_SHARED((NPAD, H), jnp.float32),
            pltpu.VMEM((8, 128), jnp.int32),
            pltpu.VMEM((8, 128), jnp.int32),
            pltpu.VMEM((128, H), jnp.float32),
            pltpu.VMEM((CW, H), jnp.float32),
            pltpu.VMEM((CW, H), jnp.float32),
            pltpu.VMEM((CW, H), jnp.float32),
            pltpu.SemaphoreType.DMA,
        ],
        compiler_params=pltpu.CompilerParams(needs_layout_passes=False, use_tc_tiling_on_sc=False),
    )(table0, colb, rowb, disx, egoT, zerosa)


@jax.jit
def kernel(features, preference, W1, b1, W2, b2, edge_index):
    row = edge_index[0]
    col = edge_index[1]
    npe = EP - E
    pad = jnp.arange(npe, dtype=jnp.int32)
    colp = jnp.concatenate([col, (pad * 16) % N])
    rowp = jnp.concatenate([row, N + (pad % 16)])
    colb = jnp.stack([colp, colp + NPAD]).reshape(2, 16, 49, 8, 128)
    rowb = rowp.reshape(16, 49, 8, 128)
    zerosa = jnp.zeros((RPT, H), jnp.float32)
    zerosf = jnp.zeros((NPAD,), jnp.float32)

    _, degp = _degree(rowb, zerosf)
    degc = degp.reshape(2, NPAD, 1)

    item_emb = _mlp(features, W1, b1, W2, b2)
    ego0 = jnp.concatenate([preference, item_emb], axis=0)
    ego0p = jnp.concatenate(
        [ego0, jnp.zeros((NPAD - N, D), jnp.float32)], axis=0)

    egoT = jnp.stack([ego0p[:, :H], ego0p[:, H:]])
    table0, disx = _table0(egoT, degc)
    out, _, _ = _propagate(table0, colb, rowb, disx, egoT, zerosa)
    final = jnp.concatenate([out[0], out[1]], axis=1)
    return final[:N_USER], final[N_USER:N]


# speedup vs baseline: 12.8652x; 1.3127x over previous
import functools

import jax
import jax.numpy as jnp
from jax import lax
from jax.experimental import pallas as pl
from jax.experimental.pallas import tpu as pltpu
from jax.experimental.pallas import tpu_sc as plsc

N_USER = 25000
N_ITEM = 25000
N = N_USER + N_ITEM
D = 64
H = 32
NPAD = 51200
E = 800000
EP = 802816
RPT = NPAD // 16
CW = 128


SEG = NPAD // 16


def _sca_body(rowb, zerosf, hists, degp, hist, rowbuf, sbuf):
    c = lax.axis_index("c")
    s = lax.axis_index("s")
    pltpu.sync_copy(zerosf, hist)
    ones = jnp.ones((16,), jnp.float32)

    def hgroup(gi, carry):
        pltpu.sync_copy(rowb.at[s, gi], rowbuf)
        for j in range(8):
            for g in range(8):
                idx = rowbuf[j, pl.ds(g * 16, 16)]
                plsc.addupdate_scatter(hist, [idx], ones)
        return carry

    lax.fori_loop(c * 25, 25 + c * 24, hgroup, 0)

    pltpu.sync_copy(hist, hists.at[c, s])
    plsc.subcore_barrier()
    for t in range(16):
        pltpu.sync_copy(hists.at[c, t, pl.ds(s * SEG, SEG)], sbuf.at[t])

    def addg(g, carry):
        tot = sbuf[0, pl.ds(g * 16, 16)]
        for t in range(1, 16):
            tot = tot + sbuf[t, pl.ds(g * 16, 16)]
        hist[pl.ds(g * 16, 16)] = tot
        return carry

    lax.fori_loop(0, SEG // 16, addg, 0)
    pltpu.sync_copy(hist.at[pl.ds(0, SEG)], degp.at[c, pl.ds(s * SEG, SEG)])


def _degree(rowb, zerosf):
    mesh = plsc.VectorSubcoreMesh(core_axis_name="c", subcore_axis_name="s")
    return pl.kernel(
        _sca_body,
        out_type=(
            jax.ShapeDtypeStruct((2, 16, NPAD), jnp.float32),
            jax.ShapeDtypeStruct((2, NPAD), jnp.float32),
        ),
        mesh=mesh,
        scratch_types=[
            pltpu.VMEM((NPAD,), jnp.float32),
            pltpu.VMEM((8, 128), jnp.int32),
            pltpu.VMEM((16, SEG), jnp.float32),
        ],
        compiler_params=pltpu.CompilerParams(needs_layout_passes=False, use_tc_tiling_on_sc=False),
    )(rowb, zerosf)


def _mlp_body(x_ref, w1_ref, b1_ref, w2_ref, b2_ref, o_ref):
    h = lax.dot_general(x_ref[...], w1_ref[...],
                        (((1,), (1,)), ((), ())),
                        preferred_element_type=jnp.float32)
    h = h + b1_ref[...]
    h = jnp.where(h >= 0, h, 0.01 * h)
    o = lax.dot_general(h, w2_ref[...],
                        (((1,), (1,)), ((), ())),
                        preferred_element_type=jnp.float32)
    o_ref[...] = o + b2_ref[...]


def _mlp(features, W1, b1, W2, b2):
    bm = 1000
    return pl.pallas_call(
        _mlp_body,
        grid=(N_ITEM // bm,),
        in_specs=[
            pl.BlockSpec((bm, 128), lambda i: (i, 0)),
            pl.BlockSpec((256, 128), lambda i: (0, 0)),
            pl.BlockSpec((1, 256), lambda i: (0, 0)),
            pl.BlockSpec((64, 256), lambda i: (0, 0)),
            pl.BlockSpec((1, 64), lambda i: (0, 0)),
        ],
        out_specs=pl.BlockSpec((bm, 64), lambda i: (i, 0)),
        out_shape=jax.ShapeDtypeStruct((N_ITEM, 64), jnp.float32),
    )(features, W1, b1.reshape(1, 256), W2, b2.reshape(1, 64))


def _table_body(ego_ref, degc_ref, table_ref, disx_ref):
    deg = degc_ref[0] + degc_ref[1]
    dis = jnp.where(deg > 0, lax.rsqrt(jnp.maximum(deg, 1e-12)), 0.0)
    disx = jnp.broadcast_to(dis, (CW, H))
    disx_ref[...] = disx
    table_ref[...] = ego_ref[0] * disx


def _table0(egoT, degc):
    nb = NPAD // CW
    return pl.pallas_call(
        _table_body,
        grid=(2, nb),
        in_specs=[
            pl.BlockSpec((1, CW, H), lambda c, i: (c, i, 0)),
            pl.BlockSpec((2, CW, 1), lambda c, i: (0, i, 0)),
        ],
        out_specs=[
            pl.BlockSpec((CW, H), lambda c, i: (c * nb + i, 0)),
            pl.BlockSpec((CW, H), lambda c, i: (i, 0)),
        ],
        out_shape=[
            jax.ShapeDtypeStruct((2 * NPAD, H), jnp.float32),
            jax.ShapeDtypeStruct((NPAD, H), jnp.float32),
        ],
    )(egoT, degc)


def _scb_body(table0, colb, rowb, disx, egoT, zerosa,
              out, table1, fsum,
              acc, colbuf, rowbuf, gbuf, accbuf, e0buf, disbuf, gsem, isem):
    c = lax.axis_index("c")
    s = lax.axis_index("s")

    def edge_pass(table_ref):
        pltpu.sync_copy(colb.at[c, s, 0], colbuf.at[0])
        pltpu.sync_copy(rowb.at[s, 0], rowbuf.at[0])
        pltpu.async_copy(table_ref.at[colbuf.at[0, 0]], gbuf.at[0], gsem)

        def group(gi, carry):
            p = jnp.bitwise_and(gi, 1)
            q = 1 - p
            gnext = jnp.minimum(gi + 1, 48)
            ic = pltpu.async_copy(colb.at[c, s, gnext], colbuf.at[q], isem)
            ir = pltpu.async_copy(rowb.at[s, gnext], rowbuf.at[q], isem)
            for j in range(8):
                if j < 7:
                    pltpu.async_copy(table_ref.at[colbuf.at[p, j + 1]],
                                     gbuf.at[(j + 1) & 1], gsem)
                else:
                    ic.wait()
                    ir.wait()

                    @pl.when(gi < 48)
                    def _nextg():
                        pltpu.async_copy(table_ref.at[colbuf.at[q, 0]],
                                         gbuf.at[0], gsem)
                pltpu.make_async_copy(table_ref.at[colbuf.at[p, j]],
                                      gbuf.at[j & 1], gsem).wait()
                pltpu.sync_copy(gbuf.at[j & 1], acc.at[rowbuf.at[p, j]],
                                add=True)
            return carry

        lax.fori_loop(0, 49, group, 0)

    def writeback(layer):
        for ch in range(RPT // CW):
            b = s * RPT + ch * CW
            pltpu.sync_copy(acc.at[pl.ds(b, CW), :], accbuf)
            if layer == 1:
                pltpu.sync_copy(egoT.at[c, pl.ds(b, CW), :], e0buf)
            else:
                pltpu.sync_copy(fsum.at[c, pl.ds(b, CW), :], e0buf)
            pltpu.sync_copy(disx.at[pl.ds(b, CW), :], disbuf)

            def rowfn(r, carry):
                for g in range(2):
                    dv = disbuf[r, pl.ds(g * 16, 16)]
                    a = accbuf[r, pl.ds(g * 16, 16)] * dv
                    f = e0buf[r, pl.ds(g * 16, 16)] + a
                    if layer == 1:
                        e0buf[r, pl.ds(g * 16, 16)] = f
                        accbuf[r, pl.ds(g * 16, 16)] = a * dv
                    else:
                        e0buf[r, pl.ds(g * 16, 16)] = f * (1.0 / 3.0)
                return carry

            lax.fori_loop(0, CW, rowfn, 0)
            if layer == 1:
                pltpu.sync_copy(e0buf, fsum.at[c, pl.ds(b, CW), :])
                pltpu.sync_copy(accbuf, table1.at[pl.ds(c * NPAD + b, CW), :])
                pltpu.sync_copy(zerosa.at[pl.ds(0, CW), :],
                                acc.at[pl.ds(b, CW), :])
            else:
                pltpu.sync_copy(e0buf, out.at[c, pl.ds(b, CW), :])

    pltpu.sync_copy(zerosa, acc.at[pl.ds(s * RPT, RPT), :])
    plsc.subcore_barrier()
    edge_pass(table0)
    plsc.subcore_barrier()
    writeback(1)
    plsc.subcore_barrier()
    edge_pass(table1)
    plsc.subcore_barrier()
    writeback(2)


def _propagate(table0, colb, rowb, disx, egoT, zerosa):
    mesh = plsc.VectorSubcoreMesh(core_axis_name="c", subcore_axis_name="s")
    return pl.kernel(
        _scb_body,
        out_type=(
            jax.ShapeDtypeStruct((2, NPAD, H), jnp.float32),
            jax.ShapeDtypeStruct((2 * NPAD, H), jnp.float32),
            jax.ShapeDtypeStruct((2, NPAD, H), jnp.float32),
        ),
        mesh=mesh,
        scratch_types=[
            pltpu.VMEM_SHARED((NPAD, H), jnp.float32),
            pltpu.VMEM((2, 8, 128), jnp.int32),
            pltpu.VMEM((2, 8, 128), jnp.int32),
            pltpu.VMEM((2, 128, H), jnp.float32),
            pltpu.VMEM((CW, H), jnp.float32),
            pltpu.VMEM((CW, H), jnp.float32),
            pltpu.VMEM((CW, H), jnp.float32),
            pltpu.SemaphoreType.DMA,
            pltpu.SemaphoreType.DMA,
        ],
        compiler_params=pltpu.CompilerParams(needs_layout_passes=False, use_tc_tiling_on_sc=False),
    )(table0, colb, rowb, disx, egoT, zerosa)


@jax.jit
def kernel(features, preference, W1, b1, W2, b2, edge_index):
    row = edge_index[0]
    col = edge_index[1]
    npe = EP - E
    pad = jnp.arange(npe, dtype=jnp.int32)
    colp = jnp.concatenate([col, (pad * 16) % N])
    rowp = jnp.concatenate([row, N + (pad % 16)])
    colb = jnp.stack([colp, colp + NPAD]).reshape(2, 16, 49, 8, 128)
    rowb = rowp.reshape(16, 49, 8, 128)
    zerosa = jnp.zeros((RPT, H), jnp.float32)
    zerosf = jnp.zeros((NPAD,), jnp.float32)

    _, degp = _degree(rowb, zerosf)
    degc = degp.reshape(2, NPAD, 1)

    item_emb = _mlp(features, W1, b1, W2, b2)
    ego0 = jnp.concatenate([preference, item_emb], axis=0)
    ego0p = jnp.concatenate(
        [ego0, jnp.zeros((NPAD - N, D), jnp.float32)], axis=0)

    egoT = jnp.stack([ego0p[:, :H], ego0p[:, H:]])
    table0, disx = _table0(egoT, degc)
    out, _, _ = _propagate(table0, colb, rowb, disx, egoT, zerosa)
    final = jnp.concatenate([out[0], out[1]], axis=1)
    return final[:N_USER], final[N_USER:N]


# speedup vs baseline: 24.8292x; 1.9299x over previous
import functools

import jax
import jax.numpy as jnp
from jax import lax
from jax.experimental import pallas as pl
from jax.experimental.pallas import tpu as pltpu
from jax.experimental.pallas import tpu_sc as plsc

N_USER = 25000
N_ITEM = 25000
N = N_USER + N_ITEM
D = 64
H = 32
NPAD = 51200
E = 800000
EP = 802816
RPT = NPAD // 16
CW = 128
SEG = NPAD // 16


def _sca_body(rowb, zerosf, hists, degp, hist, rowbuf, sbuf):
    c = lax.axis_index("c")
    s = lax.axis_index("s")
    pltpu.sync_copy(zerosf, hist)
    ones = jnp.ones((16,), jnp.float32)

    def hgroup(gi, carry):
        pltpu.sync_copy(rowb.at[s, gi], rowbuf)
        for j in range(8):
            for g in range(8):
                idx = rowbuf[j, pl.ds(g * 16, 16)]
                plsc.addupdate_scatter(hist, [idx], ones)
        return carry

    lax.fori_loop(c * 25, 25 + c * 24, hgroup, 0)

    pltpu.sync_copy(hist, hists.at[c, s])
    plsc.subcore_barrier()
    for t in range(16):
        pltpu.sync_copy(hists.at[c, t, pl.ds(s * SEG, SEG)], sbuf.at[t])

    def addg(g, carry):
        tot = sbuf[0, pl.ds(g * 16, 16)]
        for t in range(1, 16):
            tot = tot + sbuf[t, pl.ds(g * 16, 16)]
        hist[pl.ds(g * 16, 16)] = tot
        return carry

    lax.fori_loop(0, SEG // 16, addg, 0)
    pltpu.sync_copy(hist.at[pl.ds(0, SEG)], degp.at[c, pl.ds(s * SEG, SEG)])


def _degree(rowb, zerosf):
    mesh = plsc.VectorSubcoreMesh(core_axis_name="c", subcore_axis_name="s")
    return pl.kernel(
        _sca_body,
        out_type=(
            jax.ShapeDtypeStruct((2, 16, NPAD), jnp.float32),
            jax.ShapeDtypeStruct((2, NPAD), jnp.float32),
        ),
        mesh=mesh,
        scratch_types=[
            pltpu.VMEM((NPAD,), jnp.float32),
            pltpu.VMEM((8, 128), jnp.int32),
            pltpu.VMEM((16, SEG), jnp.float32),
        ],
        compiler_params=pltpu.CompilerParams(
            needs_layout_passes=False, use_tc_tiling_on_sc=False),
    )(rowb, zerosf)


def _mlp_body(x_ref, w1_ref, b1_ref, w2_ref, b2_ref, o_ref):
    h = lax.dot_general(x_ref[...], w1_ref[...],
                        (((1,), (1,)), ((), ())),
                        preferred_element_type=jnp.float32)
    h = h + b1_ref[...]
    h = jnp.where(h >= 0, h, 0.01 * h)
    o = lax.dot_general(h, w2_ref[...],
                        (((1,), (1,)), ((), ())),
                        preferred_element_type=jnp.float32)
    o_ref[...] = o + b2_ref[...]


def _mlp(features, W1, b1, W2, b2):
    bm = 1000
    return pl.pallas_call(
        _mlp_body,
        grid=(N_ITEM // bm,),
        in_specs=[
            pl.BlockSpec((bm, 128), lambda i: (i, 0)),
            pl.BlockSpec((256, 128), lambda i: (0, 0)),
            pl.BlockSpec((1, 256), lambda i: (0, 0)),
            pl.BlockSpec((64, 256), lambda i: (0, 0)),
            pl.BlockSpec((1, 64), lambda i: (0, 0)),
        ],
        out_specs=pl.BlockSpec((bm, 64), lambda i: (i, 0)),
        out_shape=jax.ShapeDtypeStruct((N_ITEM, 64), jnp.float32),
    )(features, W1, b1.reshape(1, 256), W2, b2.reshape(1, 64))


def _dis_body(degp_ref, dis_ref):
    deg = degp_ref[0] + degp_ref[1]
    dis_ref[...] = jnp.where(deg > 0, lax.rsqrt(jnp.maximum(deg, 1e-12)), 0.0)


def _dis(degp):
    return pl.pallas_call(
        _dis_body,
        out_shape=jax.ShapeDtypeStruct((400, 128), jnp.float32),
    )(degp)


def _scb_body(ego0p, colb, rowb, dis, zerosa,
              out, table0, table1, fsum,
              acc, colbuf, rowbuf, gbuf, accbuf, e0buf, dchunk, gsem, isem):
    c = lax.axis_index("c")
    s = lax.axis_index("s")

    def edge_pass(table_ref):
        pltpu.sync_copy(colb.at[s, 0], colbuf.at[0])
        pltpu.sync_copy(rowb.at[s, 0], rowbuf.at[0])
        pltpu.async_copy(table_ref.at[colbuf.at[0, 0]], gbuf.at[0], gsem)

        def group(gi, carry):
            p = jnp.bitwise_and(gi, 1)
            q = 1 - p
            gnext = jnp.minimum(gi + 1, 48)
            ic = pltpu.async_copy(colb.at[s, gnext], colbuf.at[q], isem)
            ir = pltpu.async_copy(rowb.at[s, gnext], rowbuf.at[q], isem)
            for j in range(8):
                if j < 7:
                    pltpu.async_copy(table_ref.at[colbuf.at[p, j + 1]],
                                     gbuf.at[(j + 1) & 1], gsem)
                else:
                    ic.wait()
                    ir.wait()

                    @pl.when(gi < 48)
                    def _nextg():
                        pltpu.async_copy(table_ref.at[colbuf.at[q, 0]],
                                         gbuf.at[0], gsem)
                pltpu.make_async_copy(table_ref.at[colbuf.at[p, j]],
                                      gbuf.at[j & 1], gsem).wait()
                pltpu.sync_copy(gbuf.at[j & 1], acc.at[rowbuf.at[p, j]],
                                add=True)
            return carry

        lax.fori_loop(0, 49, group, 0)

    def build_table():
        def chunk_fn(ch, carry):
            b = s * RPT + ch * CW
            pltpu.sync_copy(ego0p.at[pl.ds(b, CW), pl.ds(c * H, H)], e0buf)
            pltpu.sync_copy(dis.at[pl.ds(b, CW)], dchunk)

            def grp(g, carry2):
                d16 = dchunk[pl.ds(g * 16, 16)]
                for j in range(16):
                    r = g * 16 + j
                    dv = jnp.broadcast_to(d16[j], (16,))
                    for h in range(2):
                        e0buf[r, pl.ds(h * 16, 16)] = (
                            e0buf[r, pl.ds(h * 16, 16)] * dv)
                return carry2

            lax.fori_loop(0, CW // 16, grp, 0)
            pltpu.sync_copy(e0buf, table0.at[c, pl.ds(b, CW), :])
            return carry

        lax.fori_loop(0, RPT // CW, chunk_fn, 0)

    def writeback(layer):
        def chunk_fn(ch, carry):
            b = s * RPT + ch * CW
            pltpu.sync_copy(acc.at[pl.ds(b, CW), :], accbuf)
            if layer == 1:
                pltpu.sync_copy(ego0p.at[pl.ds(b, CW), pl.ds(c * H, H)],
                                e0buf)
            else:
                pltpu.sync_copy(fsum.at[c, pl.ds(b, CW), :], e0buf)
            pltpu.sync_copy(dis.at[pl.ds(b, CW)], dchunk)

            def grp(g, carry2):
                d16 = dchunk[pl.ds(g * 16, 16)]
                for j in range(16):
                    r = g * 16 + j
                    dv = jnp.broadcast_to(d16[j], (16,))
                    for h in range(2):
                        a = accbuf[r, pl.ds(h * 16, 16)] * dv
                        f = e0buf[r, pl.ds(h * 16, 16)] + a
                        if layer == 1:
                            e0buf[r, pl.ds(h * 16, 16)] = f
                            accbuf[r, pl.ds(h * 16, 16)] = a * dv
                        else:
                            e0buf[r, pl.ds(h * 16, 16)] = f * (1.0 / 3.0)
                return carry2

            lax.fori_loop(0, CW // 16, grp, 0)
            if layer == 1:
                pltpu.sync_copy(e0buf, fsum.at[c, pl.ds(b, CW), :])
                pltpu.sync_copy(accbuf, table1.at[c, pl.ds(b, CW), :])
                pltpu.sync_copy(zerosa.at[pl.ds(0, CW), :],
                                acc.at[pl.ds(b, CW), :])
            else:
                pltpu.sync_copy(e0buf, out.at[pl.ds(b, CW), pl.ds(c * H, H)])
            return carry

        lax.fori_loop(0, RPT // CW, chunk_fn, 0)

    pltpu.sync_copy(zerosa, acc.at[pl.ds(s * RPT, RPT), :])
    build_table()
    plsc.subcore_barrier()
    edge_pass(table0.at[c])
    plsc.subcore_barrier()
    writeback(1)
    plsc.subcore_barrier()
    edge_pass(table1.at[c])
    plsc.subcore_barrier()
    writeback(2)


def _propagate(ego0p, colb, rowb, dis, zerosa):
    mesh = plsc.VectorSubcoreMesh(core_axis_name="c", subcore_axis_name="s")
    return pl.kernel(
        _scb_body,
        out_type=(
            jax.ShapeDtypeStruct((NPAD, D), jnp.float32),
            jax.ShapeDtypeStruct((2, NPAD, H), jnp.float32),
            jax.ShapeDtypeStruct((2, NPAD, H), jnp.float32),
            jax.ShapeDtypeStruct((2, NPAD, H), jnp.float32),
        ),
        mesh=mesh,
        scratch_types=[
            pltpu.VMEM_SHARED((NPAD, H), jnp.float32),
            pltpu.VMEM((2, 8, 128), jnp.int32),
            pltpu.VMEM((2, 8, 128), jnp.int32),
            pltpu.VMEM((2, 128, H), jnp.float32),
            pltpu.VMEM((CW, H), jnp.float32),
            pltpu.VMEM((CW, H), jnp.float32),
            pltpu.VMEM((CW,), jnp.float32),
            pltpu.SemaphoreType.DMA,
            pltpu.SemaphoreType.DMA,
        ],
        compiler_params=pltpu.CompilerParams(
            needs_layout_passes=False, use_tc_tiling_on_sc=False),
    )(ego0p, colb, rowb, dis, zerosa)


@jax.jit
def kernel(features, preference, W1, b1, W2, b2, edge_index):
    row = edge_index[0]
    col = edge_index[1]
    npe = EP - E
    pad = jnp.arange(npe, dtype=jnp.int32)
    colp = jnp.concatenate([col, (pad * 16) % N])
    rowp = jnp.concatenate([row, N + (pad % 16)])
    colb = colp.reshape(16, 49, 8, 128)
    rowb = rowp.reshape(16, 49, 8, 128)
    zerosa = jnp.zeros((RPT, H), jnp.float32)
    zerosf = jnp.zeros((NPAD,), jnp.float32)

    _, degp = _degree(rowb, zerosf)
    dis = _dis(degp.reshape(2, 400, 128)).reshape(NPAD)

    item_emb = _mlp(features, W1, b1, W2, b2)
    ego0p = jnp.concatenate(
        [preference, item_emb, jnp.zeros((NPAD - N, D), jnp.float32)], axis=0)

    out, _, _, _ = _propagate(ego0p, colb, rowb, dis, zerosa)
    return out[:N_USER], out[N_USER:N]


# speedup vs baseline: 27.3588x; 1.1019x over previous
import functools

import jax
import jax.numpy as jnp
import numpy as np
from jax import lax
from jax.experimental import pallas as pl
from jax.experimental.pallas import tpu as pltpu
from jax.experimental.pallas import tpu_sc as plsc

N_USER = 25000
N_ITEM = 25000
N = N_USER + N_ITEM
D = 64
H = 32
NPAD = 51200
E = 800000
EP = 802816
RPT = NPAD // 16
CW = 128
SEG = NPAD // 16


def _sca_body(rowb, zerosf, hists, degp, hist, rowbuf, sbuf):
    c = lax.axis_index("c")
    s = lax.axis_index("s")
    pltpu.sync_copy(zerosf, hist)
    ones = jnp.ones((16,), jnp.float32)

    def hgroup(gi, carry):
        pltpu.sync_copy(rowb.at[s, gi], rowbuf)
        for j in range(8):
            for g in range(8):
                idx = rowbuf[j, pl.ds(g * 16, 16)]
                plsc.addupdate_scatter(hist, [idx], ones)
        return carry

    lax.fori_loop(c * 25, 25 + c * 24, hgroup, 0)

    pltpu.sync_copy(hist, hists.at[c, s])
    plsc.subcore_barrier()
    for t in range(16):
        pltpu.sync_copy(hists.at[c, t, pl.ds(s * SEG, SEG)], sbuf.at[t])

    def addg(g, carry):
        tot = sbuf[0, pl.ds(g * 16, 16)]
        for t in range(1, 16):
            tot = tot + sbuf[t, pl.ds(g * 16, 16)]
        hist[pl.ds(g * 16, 16)] = tot
        return carry

    lax.fori_loop(0, SEG // 16, addg, 0)
    pltpu.sync_copy(hist.at[pl.ds(0, SEG)], degp.at[c, pl.ds(s * SEG, SEG)])


def _degree(rowb, zerosf):
    mesh = plsc.VectorSubcoreMesh(core_axis_name="c", subcore_axis_name="s")
    return pl.kernel(
        _sca_body,
        out_type=(
            jax.ShapeDtypeStruct((2, 16, NPAD), jnp.float32),
            jax.ShapeDtypeStruct((2, NPAD), jnp.float32),
        ),
        mesh=mesh,
        scratch_types=[
            pltpu.VMEM((NPAD,), jnp.float32),
            pltpu.VMEM((8, 128), jnp.int32),
            pltpu.VMEM((16, SEG), jnp.float32),
        ],
        compiler_params=pltpu.CompilerParams(
            needs_layout_passes=False, use_tc_tiling_on_sc=False),
    )(rowb, zerosf)


def _mlp_body(x_ref, w1_ref, b1_ref, w2_ref, b2_ref, o_ref):
    h = lax.dot_general(x_ref[...], w1_ref[...],
                        (((1,), (1,)), ((), ())),
                        preferred_element_type=jnp.float32)
    h = h + b1_ref[...]
    h = jnp.where(h >= 0, h, 0.01 * h)
    o = lax.dot_general(h, w2_ref[...],
                        (((1,), (1,)), ((), ())),
                        preferred_element_type=jnp.float32)
    o_ref[...] = o + b2_ref[...]


def _mlp(features, W1, b1, W2, b2):
    bm = 1000
    return pl.pallas_call(
        _mlp_body,
        grid=(N_ITEM // bm,),
        in_specs=[
            pl.BlockSpec((bm, 128), lambda i: (i, 0)),
            pl.BlockSpec((256, 128), lambda i: (0, 0)),
            pl.BlockSpec((1, 256), lambda i: (0, 0)),
            pl.BlockSpec((64, 256), lambda i: (0, 0)),
            pl.BlockSpec((1, 64), lambda i: (0, 0)),
        ],
        out_specs=pl.BlockSpec((bm, 64), lambda i: (i, 0)),
        out_shape=jax.ShapeDtypeStruct((N_ITEM, 64), jnp.float32),
    )(features, W1, b1.reshape(1, 256), W2, b2.reshape(1, 64))


def _dis_body(degp_ref, dis_ref):
    deg = degp_ref[0] + degp_ref[1]
    dis_ref[...] = jnp.where(deg > 0, lax.rsqrt(jnp.maximum(deg, 1e-12)), 0.0)


def _dis(degp):
    return pl.pallas_call(
        _dis_body,
        out_shape=jax.ShapeDtypeStruct((400, 128), jnp.float32),
    )(degp)


def _scb_body(ego0p, colb, rowb, dis, zerosa,
              out, table0, table1, fsum,
              acc, colbuf, rowbuf, gbuf, accbuf, e0buf, dchunk,
              gsem, isem, ssem):
    c = lax.axis_index("c")
    s = lax.axis_index("s")

    def edge_pass(table_ref):
        pltpu.sync_copy(colb.at[s, 0], colbuf.at[0])
        pltpu.sync_copy(rowb.at[s, 0], rowbuf.at[0])
        pltpu.async_copy(table_ref.at[colbuf.at[0, 0]], gbuf.at[0], gsem)

        def group(gi, carry):
            p = jnp.bitwise_and(gi, 1)
            q = 1 - p
            gnext = jnp.minimum(gi + 1, 48)
            ic = ir = None
            for j in range(8):
                n = gi * 8 + j
                m = lax.rem(n, 3)
                mn = lax.rem(n + 1, 3)

                @pl.when(n >= 2)
                def _free():
                    pltpu.make_async_copy(gbuf.at[mn],
                                          acc.at[rowbuf.at[p, j]],
                                          ssem).wait()
                if j == 2:
                    ic = pltpu.async_copy(colb.at[s, gnext], colbuf.at[q],
                                          isem)
                    ir = pltpu.async_copy(rowb.at[s, gnext], rowbuf.at[q],
                                          isem)
                if j < 7:
                    pltpu.async_copy(table_ref.at[colbuf.at[p, j + 1]],
                                     gbuf.at[mn], gsem)
                else:
                    ic.wait()
                    ir.wait()

                    @pl.when(gi < 48)
                    def _nextg():
                        pltpu.async_copy(table_ref.at[colbuf.at[q, 0]],
                                         gbuf.at[mn], gsem)
                pltpu.make_async_copy(table_ref.at[colbuf.at[p, j]],
                                      gbuf.at[m], gsem).wait()
                pltpu.async_copy(gbuf.at[m], acc.at[rowbuf.at[p, j]],
                                 ssem, add=True)
            return carry

        lax.fori_loop(0, 49, group, 0)
        for _ in range(2):
            pltpu.make_async_copy(gbuf.at[0], acc.at[rowbuf.at[1, 0]],
                                  ssem).wait()

    def build_table():
        def chunk_fn(ch, carry):
            b = s * RPT + ch * CW
            pltpu.sync_copy(ego0p.at[pl.ds(b, CW), pl.ds(c * H, H)], e0buf)
            pltpu.sync_copy(dis.at[pl.ds(b, CW)], dchunk)

            def grp(g, carry2):
                d16 = dchunk[pl.ds(g * 16, 16)]
                for j in range(16):
                    r = g * 16 + j
                    dv = jnp.broadcast_to(d16[j], (16,))
                    for h in range(2):
                        e0buf[r, pl.ds(h * 16, 16)] = (
                            e0buf[r, pl.ds(h * 16, 16)] * dv)
                return carry2

            lax.fori_loop(0, CW // 16, grp, 0)
            pltpu.sync_copy(e0buf, table0.at[c, pl.ds(b, CW), :])
            return carry

        lax.fori_loop(0, RPT // CW, chunk_fn, 0)

    def writeback(layer):
        def chunk_fn(ch, carry):
            b = s * RPT + ch * CW
            pltpu.sync_copy(acc.at[pl.ds(b, CW), :], accbuf)
            if layer == 1:
                pltpu.sync_copy(ego0p.at[pl.ds(b, CW), pl.ds(c * H, H)],
                                e0buf)
            else:
                pltpu.sync_copy(fsum.at[c, pl.ds(b, CW), :], e0buf)
            pltpu.sync_copy(dis.at[pl.ds(b, CW)], dchunk)

            def grp(g, carry2):
                d16 = dchunk[pl.ds(g * 16, 16)]
                for j in range(16):
                    r = g * 16 + j
                    dv = jnp.broadcast_to(d16[j], (16,))
                    for h in range(2):
                        a = accbuf[r, pl.ds(h * 16, 16)] * dv
                        f = e0buf[r, pl.ds(h * 16, 16)] + a
                        if layer == 1:
                            e0buf[r, pl.ds(h * 16, 16)] = f
                            accbuf[r, pl.ds(h * 16, 16)] = a * dv
                        else:
                            e0buf[r, pl.ds(h * 16, 16)] = f * (1.0 / 3.0)
                return carry2

            lax.fori_loop(0, CW // 16, grp, 0)
            if layer == 1:
                pltpu.sync_copy(e0buf, fsum.at[c, pl.ds(b, CW), :])
                pltpu.sync_copy(accbuf, table1.at[c, pl.ds(b, CW), :])
                pltpu.sync_copy(zerosa.at[pl.ds(0, CW), :],
                                acc.at[pl.ds(b, CW), :])
            else:
                pltpu.sync_copy(e0buf, out.at[pl.ds(b, CW), pl.ds(c * H, H)])
            return carry

        lax.fori_loop(0, RPT // CW, chunk_fn, 0)

    pltpu.sync_copy(zerosa, acc.at[pl.ds(s * RPT, RPT), :])
    build_table()
    plsc.subcore_barrier()
    edge_pass(table0.at[c])
    plsc.subcore_barrier()
    writeback(1)
    plsc.subcore_barrier()
    edge_pass(table1.at[c])
    plsc.subcore_barrier()
    writeback(2)


def _propagate(ego0p, colb, rowb, dis, zerosa):
    mesh = plsc.VectorSubcoreMesh(core_axis_name="c", subcore_axis_name="s")
    return pl.kernel(
        _scb_body,
        out_type=(
            jax.ShapeDtypeStruct((NPAD, D), jnp.float32),
            jax.ShapeDtypeStruct((2, NPAD, H), jnp.float32),
            jax.ShapeDtypeStruct((2, NPAD, H), jnp.float32),
            jax.ShapeDtypeStruct((2, NPAD, H), jnp.float32),
        ),
        mesh=mesh,
        scratch_types=[
            pltpu.VMEM_SHARED((NPAD, H), jnp.float32),
            pltpu.VMEM((2, 8, 128), jnp.int32),
            pltpu.VMEM((2, 8, 128), jnp.int32),
            pltpu.VMEM((3, 128, H), jnp.float32),
            pltpu.VMEM((CW, H), jnp.float32),
            pltpu.VMEM((CW, H), jnp.float32),
            pltpu.VMEM((CW,), jnp.float32),
            pltpu.SemaphoreType.DMA,
            pltpu.SemaphoreType.DMA,
            pltpu.SemaphoreType.DMA,
        ],
        compiler_params=pltpu.CompilerParams(
            needs_layout_passes=False, use_tc_tiling_on_sc=False),
    )(ego0p, colb, rowb, dis, zerosa)


@jax.jit
def kernel(features, preference, W1, b1, W2, b2, edge_index):
    row = edge_index[0]
    col = edge_index[1]
    npe = EP - E
    pad = np.arange(npe)
    colp = jnp.concatenate(
        [col, jnp.asarray((pad * 16) % N, jnp.int32)])
    rowp = jnp.concatenate(
        [row, jnp.asarray(N + (pad % 16), jnp.int32)])
    colb = colp.reshape(16, 49, 8, 128)
    rowb = rowp.reshape(16, 49, 8, 128)
    zerosa = jnp.zeros((RPT, H), jnp.float32)
    zerosf = jnp.zeros((NPAD,), jnp.float32)

    _, degp = _degree(rowb, zerosf)
    dis = _dis(degp.reshape(2, 400, 128)).reshape(NPAD)

    item_emb = _mlp(features, W1, b1, W2, b2)
    ego0p = jnp.concatenate(
        [preference, item_emb, jnp.zeros((NPAD - N, D), jnp.float32)], axis=0)

    out, _, _, _ = _propagate(ego0p, colb, rowb, dis, zerosa)
    return out[:N_USER], out[N_USER:N]
